# baseline (speedup 1.0000x reference)
import functools

import jax
import jax.numpy as jnp
from jax import lax
from jax.experimental import pallas as pl
from jax.experimental.pallas import tpu as pltpu
from jax.experimental.pallas import tpu_sc as plsc

N = 10000
D = 128
E = 320000

NC = 2
NS = 16
NW = NC * NS
CH = 128
NCH = -(-((E + NW * CH - 1) // (NW * CH)) // 2) * 2
NCH_A = 56
NCH_B = 2 * NCH - NCH_A
EPAD = NW * NCH * CH - E
N_ACC = ((N + NS * 8 - 1) // (NS * 8)) * NS * 8
RPS = N_ACC // NS

BR = 400
GRID = N // BR

_mesh = plsc.VectorSubcoreMesh(
    core_axis_name="c", subcore_axis_name="s", num_cores=NC, num_subcores=NS)


def _zero_chunks(total, step):
    out = []
    k = 0
    while k < total:
        out.append((k, min(step, total - k)))
        k += step
    return out


@functools.partial(
    pl.kernel,
    out_type=jax.ShapeDtypeStruct((NC, N_ACC, D), jnp.float32),
    mesh=_mesh,
    scratch_types=[
        pltpu.VMEM((NCH, CH), jnp.int32),
        pltpu.VMEM((CH, D), jnp.float32),
        pltpu.VMEM_SHARED((N_ACC, D), jnp.float32),
    ],
)
def _sc_deg(dstw, z128, o128, deg_out, dst_idx, vals, degs):
    c = lax.axis_index("c")
    s = lax.axis_index("s")
    w = s * NC + c
    pltpu.sync_copy(z128, vals)
    r0 = s * RPS
    for k, sz in _zero_chunks(RPS, CH):
        pltpu.sync_copy(vals.at[pl.ds(0, sz)], degs.at[pl.ds(r0 + k, sz)])
    pltpu.sync_copy(o128, vals)
    plsc.subcore_barrier()
    pltpu.sync_copy(dstw.at[pl.ds(w * NCH, NCH)], dst_idx)

    def chunk(i, carry):
        pltpu.sync_copy(vals, degs.at[dst_idx.at[i]], add=True)
        return carry

    lax.fori_loop(0, NCH, chunk, 0)
    plsc.subcore_barrier()
    pltpu.sync_copy(degs.at[pl.ds(r0, RPS)], deg_out.at[c].at[pl.ds(r0, RPS)])


@functools.partial(
    pl.kernel,
    out_type=jax.ShapeDtypeStruct((NC, N_ACC, D), jnp.float32),
    mesh=_mesh,
    scratch_types=[
        pltpu.VMEM((NCH_B, CH), jnp.int32),
        pltpu.VMEM((NCH_B, CH), jnp.int32),
        pltpu.VMEM((CH, D), jnp.float32),
        pltpu.VMEM_SHARED((N_ACC, D), jnp.float32),
        pltpu.SemaphoreType.DMA,
    ],
)
def _sc_agg(p_hbm, srcw, dstw, agg_out,
            src_idx, dst_idx, rows, acc, sem):
    c = lax.axis_index("c")
    s = lax.axis_index("s")

    def zstore(k, carry):
        rows[k // 8, pl.ds((k % 8) * 16, 16)] = jnp.zeros((16,), jnp.float32)
        return carry

    lax.fori_loop(0, CH * 8, zstore, 0)
    r0 = s * RPS
    for k, sz in _zero_chunks(RPS, CH):
        pltpu.sync_copy(rows.at[pl.ds(0, sz)], acc.at[pl.ds(r0 + k, sz)])
    plsc.subcore_barrier()

    def run(nch, base):
        pltpu.sync_copy(srcw.at[pl.ds(base, nch)], src_idx.at[pl.ds(0, nch)])
        pltpu.sync_copy(dstw.at[pl.ds(base, nch)], dst_idx.at[pl.ds(0, nch)])

        def chunk(i, carry):
            pltpu.async_copy(p_hbm.at[src_idx.at[i]], rows, sem).wait()
            pltpu.sync_copy(rows, acc.at[dst_idx.at[i]], add=True)
            return carry

        lax.fori_loop(0, nch, chunk, 0)

    @pl.when(c == 0)
    def _():
        run(NCH_A, s * NCH_A)

    @pl.when(c == 1)
    def _():
        run(NCH_B, NS * NCH_A + s * NCH_B)

    plsc.subcore_barrier()
    pltpu.sync_copy(acc.at[pl.ds(r0, RPS)], agg_out.at[c].at[pl.ds(r0, RPS)])


def _mm1_body(x_ref, w_ref, p_ref, q_ref):
    pq = jax.lax.dot_general(
        x_ref[...], w_ref[...], (((1,), (0,)), ((), ())),
        precision=lax.Precision.HIGHEST, preferred_element_type=jnp.float32)
    p_ref[...] = pq[:, :D]
    q_ref[...] = pq[:, D:]


_mm1 = pl.pallas_call(
    _mm1_body,
    grid=(GRID,),
    in_specs=[
        pl.BlockSpec((BR, D), lambda i: (i, 0)),
        pl.BlockSpec((D, 2 * D), lambda i: (0, 0)),
    ],
    out_specs=[
        pl.BlockSpec((BR, D), lambda i: (i, 0)),
        pl.BlockSpec((BR, D), lambda i: (i, 0)),
    ],
    out_shape=[jax.ShapeDtypeStruct((N, D), jnp.float32),
               jax.ShapeDtypeStruct((N, D), jnp.float32)],
)


def _combine2_body(agg_ref, deg_ref, q_ref, b_ref, w_ref, p2_ref, q2_ref):
    a = agg_ref[0] + agg_ref[1]
    deg = deg_ref[0, :, :1] + deg_ref[1, :, :1]
    mean = a / jnp.maximum(deg, 1.0)
    h = jnp.maximum(mean + b_ref[...] + q_ref[...], 0.0)
    pq = jax.lax.dot_general(
        h, w_ref[...], (((1,), (0,)), ((), ())),
        precision=lax.Precision.HIGHEST, preferred_element_type=jnp.float32)
    p2_ref[...] = pq[:, :D]
    q2_ref[...] = pq[:, D:]


_combine2 = pl.pallas_call(
    _combine2_body,
    grid=(GRID,),
    in_specs=[
        pl.BlockSpec((NC, BR, D), lambda i: (0, i, 0)),
        pl.BlockSpec((NC, BR, D), lambda i: (0, i, 0)),
        pl.BlockSpec((BR, D), lambda i: (i, 0)),
        pl.BlockSpec((1, D), lambda i: (0, 0)),
        pl.BlockSpec((D, 2 * D), lambda i: (0, 0)),
    ],
    out_specs=[
        pl.BlockSpec((BR, D), lambda i: (i, 0)),
        pl.BlockSpec((BR, D), lambda i: (i, 0)),
    ],
    out_shape=[jax.ShapeDtypeStruct((N, D), jnp.float32),
               jax.ShapeDtypeStruct((N, D), jnp.float32)],
)


def _final_body(agg_ref, deg_ref, q_ref, b_ref, o_ref):
    a = agg_ref[0] + agg_ref[1]
    deg = deg_ref[0, :, :1] + deg_ref[1, :, :1]
    mean = a / jnp.maximum(deg, 1.0)
    o_ref[...] = jax.nn.sigmoid(mean + b_ref[...] + q_ref[...])


_final = pl.pallas_call(
    _final_body,
    grid=(GRID,),
    in_specs=[
        pl.BlockSpec((NC, BR, D), lambda i: (0, i, 0)),
        pl.BlockSpec((NC, BR, D), lambda i: (0, i, 0)),
        pl.BlockSpec((BR, D), lambda i: (i, 0)),
        pl.BlockSpec((1, D), lambda i: (0, 0)),
    ],
    out_specs=pl.BlockSpec((BR, D), lambda i: (i, 0)),
    out_shape=jax.ShapeDtypeStruct((N, D), jnp.float32),
)


def kernel(x, edge_index, edge_attr, Wl1, bl1, Wr1, Wl2, bl2, Wr2):
    src = edge_index[0]
    dst = edge_index[1]
    srcw = jnp.concatenate(
        [src, jnp.zeros((EPAD,), jnp.int32)]).reshape(NW * NCH, CH)
    dstw = jnp.concatenate(
        [dst, jnp.full((EPAD,), N, jnp.int32)]).reshape(NW * NCH, CH)
    z128 = jnp.zeros((CH, D), jnp.float32)
    o128 = jnp.ones((CH, D), jnp.float32)
    wcat1 = jnp.concatenate([Wl1.T, Wr1.T], axis=1)
    wcat2 = jnp.concatenate([Wl2.T, Wr2.T], axis=1)
    b1 = bl1.reshape(1, D)
    b2 = bl2.reshape(1, D)

    p1, q1 = _mm1(x, wcat1)
    degp = _sc_deg(dstw, z128, o128)
    agg1 = _sc_agg(p1, srcw, dstw)
    p2, q2 = _combine2(agg1, degp, q1, b1, wcat2)
    agg2 = _sc_agg(p2, srcw, dstw)
    return _final(agg2, degp, q2, b2)


# speedup vs baseline: 1.1508x; 1.1508x over previous
import functools

import jax
import jax.numpy as jnp
from jax import lax
from jax.experimental import pallas as pl
from jax.experimental.pallas import tpu as pltpu
from jax.experimental.pallas import tpu_sc as plsc

N = 10000
D = 128
E = 320000

NC = 2
NS = 16
NW = NC * NS
CH = 128
NCH = -(-((E + NW * CH - 1) // (NW * CH)) // 2) * 2
NCH_A = 104
NCH_B = 2 * NCH - NCH_A
NCH_MAX = max(NCH_A, NCH_B)
EPAD = NW * NCH * CH - E
N_ACC = ((N + NS * 8 - 1) // (NS * 8)) * NS * 8
RPS = N_ACC // NS

BR = 400
GRID = N // BR

_mesh = plsc.VectorSubcoreMesh(
    core_axis_name="c", subcore_axis_name="s", num_cores=NC, num_subcores=NS)


def _zero_chunks(total, step):
    out = []
    k = 0
    while k < total:
        out.append((k, min(step, total - k)))
        k += step
    return out


@functools.partial(
    pl.kernel,
    out_type=jax.ShapeDtypeStruct((NC, N_ACC, D), jnp.float32),
    mesh=_mesh,
    scratch_types=[
        pltpu.VMEM((NCH, CH), jnp.int32),
        pltpu.VMEM((CH, D), jnp.float32),
        pltpu.VMEM_SHARED((N_ACC, D), jnp.float32),
    ],
)
def _sc_deg(dstw, z128, o128, deg_out, dst_idx, vals, degs):
    c = lax.axis_index("c")
    s = lax.axis_index("s")
    w = s * NC + c
    pltpu.sync_copy(z128, vals)
    r0 = s * RPS
    for k, sz in _zero_chunks(RPS, CH):
        pltpu.sync_copy(vals.at[pl.ds(0, sz)], degs.at[pl.ds(r0 + k, sz)])
    pltpu.sync_copy(o128, vals)
    plsc.subcore_barrier()
    pltpu.sync_copy(dstw.at[pl.ds(w * NCH, NCH)], dst_idx)

    def chunk(i, carry):
        pltpu.sync_copy(vals, degs.at[dst_idx.at[i]], add=True)
        return carry

    lax.fori_loop(0, NCH, chunk, 0)
    plsc.subcore_barrier()
    pltpu.sync_copy(degs.at[pl.ds(r0, RPS)], deg_out.at[c].at[pl.ds(r0, RPS)])


@functools.partial(
    pl.kernel,
    out_type=jax.ShapeDtypeStruct((NC, N_ACC, D), jnp.float32),
    mesh=_mesh,
    scratch_types=[
        pltpu.VMEM((NCH_MAX, CH), jnp.int32),
        pltpu.VMEM((NCH_MAX, CH), jnp.int32),
        pltpu.VMEM((CH, D), jnp.float32),
        pltpu.VMEM_SHARED((N_ACC, D), jnp.float32),
        pltpu.SemaphoreType.DMA,
    ],
)
def _sc_agg(p_hbm, srcw, dstw, agg_out,
            src_idx, dst_idx, rows, acc, sem):
    c = lax.axis_index("c")
    s = lax.axis_index("s")

    def zstore(k, carry):
        rows[k // 8, pl.ds((k % 8) * 16, 16)] = jnp.zeros((16,), jnp.float32)
        return carry

    lax.fori_loop(0, CH * 8, zstore, 0)
    r0 = s * RPS
    for k, sz in _zero_chunks(RPS, CH):
        pltpu.sync_copy(rows.at[pl.ds(0, sz)], acc.at[pl.ds(r0 + k, sz)])
    plsc.subcore_barrier()

    def run(nch, base):
        pltpu.sync_copy(srcw.at[pl.ds(base, nch)], src_idx.at[pl.ds(0, nch)])
        pltpu.sync_copy(dstw.at[pl.ds(base, nch)], dst_idx.at[pl.ds(0, nch)])

        def chunk(i, carry):
            pltpu.async_copy(p_hbm.at[src_idx.at[i]], rows, sem).wait()
            pltpu.sync_copy(rows, acc.at[dst_idx.at[i]], add=True)
            return carry

        lax.fori_loop(0, nch, chunk, 0)

    @pl.when(c == 0)
    def _():
        run(NCH_A, s * NCH_A)

    @pl.when(c == 1)
    def _():
        run(NCH_B, NS * NCH_A + s * NCH_B)

    plsc.subcore_barrier()
    pltpu.sync_copy(acc.at[pl.ds(r0, RPS)], agg_out.at[c].at[pl.ds(r0, RPS)])


def _mm1_body(x_ref, w_ref, p_ref, q_ref):
    pq = jax.lax.dot_general(
        x_ref[...], w_ref[...], (((1,), (0,)), ((), ())),
        precision=lax.Precision.HIGHEST, preferred_element_type=jnp.float32)
    p_ref[...] = pq[:, :D]
    q_ref[...] = pq[:, D:]


_mm1 = pl.pallas_call(
    _mm1_body,
    grid=(GRID,),
    in_specs=[
        pl.BlockSpec((BR, D), lambda i: (i, 0)),
        pl.BlockSpec((D, 2 * D), lambda i: (0, 0)),
    ],
    out_specs=[
        pl.BlockSpec((BR, D), lambda i: (i, 0)),
        pl.BlockSpec((BR, D), lambda i: (i, 0)),
    ],
    out_shape=[jax.ShapeDtypeStruct((N, D), jnp.float32),
               jax.ShapeDtypeStruct((N, D), jnp.float32)],
)


def _combine2_body(agg_ref, deg_ref, q_ref, b_ref, w_ref, p2_ref, q2_ref):
    a = agg_ref[0] + agg_ref[1]
    deg = deg_ref[0, :, :1] + deg_ref[1, :, :1]
    mean = a / jnp.maximum(deg, 1.0)
    h = jnp.maximum(mean + b_ref[...] + q_ref[...], 0.0)
    pq = jax.lax.dot_general(
        h, w_ref[...], (((1,), (0,)), ((), ())),
        precision=lax.Precision.HIGHEST, preferred_element_type=jnp.float32)
    p2_ref[...] = pq[:, :D]
    q2_ref[...] = pq[:, D:]


_combine2 = pl.pallas_call(
    _combine2_body,
    grid=(GRID,),
    in_specs=[
        pl.BlockSpec((NC, BR, D), lambda i: (0, i, 0)),
        pl.BlockSpec((NC, BR, D), lambda i: (0, i, 0)),
        pl.BlockSpec((BR, D), lambda i: (i, 0)),
        pl.BlockSpec((1, D), lambda i: (0, 0)),
        pl.BlockSpec((D, 2 * D), lambda i: (0, 0)),
    ],
    out_specs=[
        pl.BlockSpec((BR, D), lambda i: (i, 0)),
        pl.BlockSpec((BR, D), lambda i: (i, 0)),
    ],
    out_shape=[jax.ShapeDtypeStruct((N, D), jnp.float32),
               jax.ShapeDtypeStruct((N, D), jnp.float32)],
)


def _final_body(agg_ref, deg_ref, q_ref, b_ref, o_ref):
    a = agg_ref[0] + agg_ref[1]
    deg = deg_ref[0, :, :1] + deg_ref[1, :, :1]
    mean = a / jnp.maximum(deg, 1.0)
    o_ref[...] = jax.nn.sigmoid(mean + b_ref[...] + q_ref[...])


_final = pl.pallas_call(
    _final_body,
    grid=(GRID,),
    in_specs=[
        pl.BlockSpec((NC, BR, D), lambda i: (0, i, 0)),
        pl.BlockSpec((NC, BR, D), lambda i: (0, i, 0)),
        pl.BlockSpec((BR, D), lambda i: (i, 0)),
        pl.BlockSpec((1, D), lambda i: (0, 0)),
    ],
    out_specs=pl.BlockSpec((BR, D), lambda i: (i, 0)),
    out_shape=jax.ShapeDtypeStruct((N, D), jnp.float32),
)


def kernel(x, edge_index, edge_attr, Wl1, bl1, Wr1, Wl2, bl2, Wr2):
    src = edge_index[0]
    dst = edge_index[1]
    srcw = jnp.concatenate(
        [src, jnp.zeros((EPAD,), jnp.int32)]).reshape(NW * NCH, CH)
    dstw = jnp.concatenate(
        [dst, jnp.full((EPAD,), N, jnp.int32)]).reshape(NW * NCH, CH)
    z128 = jnp.zeros((CH, D), jnp.float32)
    o128 = jnp.ones((CH, D), jnp.float32)
    wcat1 = jnp.concatenate([Wl1.T, Wr1.T], axis=1)
    wcat2 = jnp.concatenate([Wl2.T, Wr2.T], axis=1)
    b1 = bl1.reshape(1, D)
    b2 = bl2.reshape(1, D)

    p1, q1 = _mm1(x, wcat1)
    degp = _sc_deg(dstw, z128, o128)
    agg1 = _sc_agg(p1, srcw, dstw)
    p2, q2 = _combine2(agg1, degp, q1, b1, wcat2)
    agg2 = _sc_agg(p2, srcw, dstw)
    return _final(agg2, degp, q2, b2)


# speedup vs baseline: 1.2118x; 1.0530x over previous
import functools

import jax
import jax.numpy as jnp
from jax import lax
from jax.experimental import pallas as pl
from jax.experimental.pallas import tpu as pltpu
from jax.experimental.pallas import tpu_sc as plsc

N = 10000
D = 128
E = 320000

NC = 2
NS = 16
NW = NC * NS
CH = 128
NCH = 80
EPAD = NW * NCH * CH - E
N_ACC = ((N + NS * 8 - 1) // (NS * 8)) * NS * 8
RPS = N_ACC // NS

BR = 400
GRID = N // BR

_mesh = plsc.VectorSubcoreMesh(
    core_axis_name="c", subcore_axis_name="s", num_cores=NC, num_subcores=NS)


def _zero_chunks(total, step):
    out = []
    k = 0
    while k < total:
        out.append((k, min(step, total - k)))
        k += step
    return out


def _fill_rows(ref, nrows, value):
    def body(k, carry):
        ref[k // 8, pl.ds((k % 8) * 16, 16)] = jnp.full((16,), value,
                                                        jnp.float32)
        return carry

    lax.fori_loop(0, nrows * 8, body, 0)


def _load_idx(packed, w, idx):
    half = NCH // 2
    pltpu.sync_copy(packed.at[w], idx.at[pl.ds(half, half)])

    def conv(k, carry):
        m = k // 8
        g = k % 8
        v = idx[half + m, pl.ds(g * 16, 16)]
        idx[2 * m, pl.ds(g * 16, 16)] = jnp.bitwise_and(v, 0xFFFF)
        idx[2 * m + 1, pl.ds(g * 16, 16)] = lax.shift_right_logical(v, 16)
        return carry

    lax.fori_loop(0, half * 8, conv, 0)


@functools.partial(
    pl.kernel,
    out_type=jax.ShapeDtypeStruct((NC, N_ACC, D), jnp.float32),
    mesh=_mesh,
    scratch_types=[
        pltpu.VMEM((NCH, CH), jnp.int32),
        pltpu.VMEM((CH, D), jnp.float32),
        pltpu.VMEM_SHARED((N_ACC, D), jnp.float32),
    ],
)
def _sc_deg(dstp, deg_out, dst_idx, vals, degs):
    c = lax.axis_index("c")
    s = lax.axis_index("s")
    w = s * NC + c
    _load_idx(dstp, w, dst_idx)
    _fill_rows(vals, CH, 0.0)
    r0 = s * RPS
    for k, sz in _zero_chunks(RPS, CH):
        pltpu.sync_copy(vals.at[pl.ds(0, sz)], degs.at[pl.ds(r0 + k, sz)])
    _fill_rows(vals, CH, 1.0)
    plsc.subcore_barrier()

    def chunk(i, carry):
        pltpu.sync_copy(vals, degs.at[dst_idx.at[i]], add=True)
        return carry

    lax.fori_loop(0, NCH, chunk, 0)
    plsc.subcore_barrier()
    pltpu.sync_copy(degs.at[pl.ds(r0, RPS)], deg_out.at[c].at[pl.ds(r0, RPS)])


NSLOT = 2


@functools.partial(
    pl.kernel,
    out_type=jax.ShapeDtypeStruct((NC, N_ACC, D), jnp.float32),
    mesh=_mesh,
    scratch_types=[
        pltpu.VMEM((NCH, CH), jnp.int32),
        pltpu.VMEM((NCH // 2, CH), jnp.int32),
        pltpu.VMEM((2, CH), jnp.int32),
        pltpu.VMEM((CH, D), jnp.float32),
        pltpu.VMEM((CH, D), jnp.float32),
        pltpu.VMEM_SHARED((N_ACC, D), jnp.float32),
        pltpu.SemaphoreType.DMA,
        pltpu.SemaphoreType.DMA,
    ],
)
def _sc_agg(p_hbm, srcp, dstp, agg_out,
            src_idx, dstp_v, idx2, rows0, rows1, acc,
            sem0, sem1):
    c = lax.axis_index("c")
    s = lax.axis_index("s")
    w = s * NC + c
    slots = ((rows0, sem0), (rows1, sem1))
    _load_idx(srcp, w, src_idx)
    pltpu.sync_copy(dstp.at[w], dstp_v)
    _fill_rows(rows0, CH, 0.0)
    r0 = s * RPS
    for k, sz in _zero_chunks(RPS, CH):
        pltpu.sync_copy(rows0.at[pl.ds(0, sz)], acc.at[pl.ds(r0 + k, sz)])
    plsc.subcore_barrier()

    for b, (rows, sem) in enumerate(slots):
        pltpu.async_copy(p_hbm.at[src_idx.at[b]], rows, sem)

    def quad(q, carry):
        def wident(g, carry2):
            v = dstp_v[q, pl.ds(g * 16, 16)]
            idx2[0, pl.ds(g * 16, 16)] = jnp.bitwise_and(v, 0xFFFF)
            idx2[1, pl.ds(g * 16, 16)] = lax.shift_right_logical(v, 16)
            return carry2

        lax.fori_loop(0, 8, wident, 0)
        for b, (rows, sem) in enumerate(slots):
            i = NSLOT * q + b
            pltpu.make_async_copy(p_hbm.at[src_idx.at[i]], rows, sem).wait()
            pltpu.sync_copy(rows, acc.at[idx2.at[b]], add=True)

            @pl.when(i + NSLOT < NCH)
            def _():
                pltpu.async_copy(p_hbm.at[src_idx.at[i + NSLOT]], rows, sem)
        return carry

    lax.fori_loop(0, NCH // NSLOT, quad, 0)
    plsc.subcore_barrier()
    pltpu.sync_copy(acc.at[pl.ds(r0, RPS)], agg_out.at[c].at[pl.ds(r0, RPS)])


def _mm1_body(x_ref, w_ref, p_ref, q_ref):
    pq = jax.lax.dot_general(
        x_ref[...], w_ref[...], (((1,), (0,)), ((), ())),
        precision=lax.Precision.HIGHEST, preferred_element_type=jnp.float32)
    p_ref[...] = pq[:, :D]
    q_ref[...] = pq[:, D:]


_mm1 = pl.pallas_call(
    _mm1_body,
    grid=(GRID,),
    in_specs=[
        pl.BlockSpec((BR, D), lambda i: (i, 0)),
        pl.BlockSpec((D, 2 * D), lambda i: (0, 0)),
    ],
    out_specs=[
        pl.BlockSpec((BR, D), lambda i: (i, 0)),
        pl.BlockSpec((BR, D), lambda i: (i, 0)),
    ],
    out_shape=[jax.ShapeDtypeStruct((N, D), jnp.float32),
               jax.ShapeDtypeStruct((N, D), jnp.float32)],
)


def _combine2_body(agg_ref, deg_ref, q_ref, b_ref, w_ref, p2_ref, q2_ref):
    a = agg_ref[0] + agg_ref[1]
    deg = deg_ref[0, :, :1] + deg_ref[1, :, :1]
    mean = a / jnp.maximum(deg, 1.0)
    h = jnp.maximum(mean + b_ref[...] + q_ref[...], 0.0)
    pq = jax.lax.dot_general(
        h, w_ref[...], (((1,), (0,)), ((), ())),
        precision=lax.Precision.HIGHEST, preferred_element_type=jnp.float32)
    p2_ref[...] = pq[:, :D]
    q2_ref[...] = pq[:, D:]


_combine2 = pl.pallas_call(
    _combine2_body,
    grid=(GRID,),
    in_specs=[
        pl.BlockSpec((NC, BR, D), lambda i: (0, i, 0)),
        pl.BlockSpec((NC, BR, D), lambda i: (0, i, 0)),
        pl.BlockSpec((BR, D), lambda i: (i, 0)),
        pl.BlockSpec((1, D), lambda i: (0, 0)),
        pl.BlockSpec((D, 2 * D), lambda i: (0, 0)),
    ],
    out_specs=[
        pl.BlockSpec((BR, D), lambda i: (i, 0)),
        pl.BlockSpec((BR, D), lambda i: (i, 0)),
    ],
    out_shape=[jax.ShapeDtypeStruct((N, D), jnp.float32),
               jax.ShapeDtypeStruct((N, D), jnp.float32)],
)


def _final_body(agg_ref, deg_ref, q_ref, b_ref, o_ref):
    a = agg_ref[0] + agg_ref[1]
    deg = deg_ref[0, :, :1] + deg_ref[1, :, :1]
    mean = a / jnp.maximum(deg, 1.0)
    o_ref[...] = jax.nn.sigmoid(mean + b_ref[...] + q_ref[...])


_final = pl.pallas_call(
    _final_body,
    grid=(GRID,),
    in_specs=[
        pl.BlockSpec((NC, BR, D), lambda i: (0, i, 0)),
        pl.BlockSpec((NC, BR, D), lambda i: (0, i, 0)),
        pl.BlockSpec((BR, D), lambda i: (i, 0)),
        pl.BlockSpec((1, D), lambda i: (0, 0)),
    ],
    out_specs=pl.BlockSpec((BR, D), lambda i: (i, 0)),
    out_shape=jax.ShapeDtypeStruct((N, D), jnp.float32),
)


def kernel(x, edge_index, edge_attr, Wl1, bl1, Wr1, Wl2, bl2, Wr2):
    src = edge_index[0]
    dst = edge_index[1]
    def pack16(a, fill):
        a3 = jnp.concatenate(
            [a, jnp.full((EPAD,), fill, jnp.int32)]).reshape(NW, NCH, CH)
        return a3[:, 0::2, :] | (a3[:, 1::2, :] << 16)

    srcp = pack16(src, 0)
    dstp = pack16(dst, N)
    wcat1 = jnp.concatenate([Wl1.T, Wr1.T], axis=1)
    wcat2 = jnp.concatenate([Wl2.T, Wr2.T], axis=1)
    b1 = bl1.reshape(1, D)
    b2 = bl2.reshape(1, D)

    p1, q1 = _mm1(x, wcat1)
    degp = _sc_deg(dstp)
    agg1 = _sc_agg(p1, srcp, dstp)
    p2, q2 = _combine2(agg1, degp, q1, b1, wcat2)
    agg2 = _sc_agg(p2, srcp, dstp)
    return _final(agg2, degp, q2, b2)


# speedup vs baseline: 1.6051x; 1.3245x over previous
import functools

import jax
import jax.numpy as jnp
from jax import lax
from jax.experimental import pallas as pl
from jax.experimental.pallas import tpu as pltpu
from jax.experimental.pallas import tpu_sc as plsc

N = 10000
D = 128
E = 320000

NC = 2
NS = 16
NW = NC * NS
CH = 128
NCH = (E + NW * CH - 1) // (NW * CH)
EPAD = NW * NCH * CH - E
N_ACC = ((N + NS * 8 - 1) // (NS * 8)) * NS * 8
RPS = N_ACC // NS

BR = 400
GRID = N // BR

_mesh = plsc.VectorSubcoreMesh(
    core_axis_name="c", subcore_axis_name="s", num_cores=NC, num_subcores=NS)


def _zero_chunks(total, step):
    out = []
    k = 0
    while k < total:
        out.append((k, min(step, total - k)))
        k += step
    return out


@functools.partial(
    pl.kernel,
    out_type=jax.ShapeDtypeStruct((NC, N_ACC, D), jnp.float32),
    mesh=_mesh,
    scratch_types=[
        pltpu.VMEM((NCH, CH), jnp.int32),
        pltpu.VMEM((CH, D), jnp.float32),
        pltpu.VMEM_SHARED((N_ACC, D), jnp.float32),
    ],
)
def _sc_deg(dstw, z128, o128, deg_out, dst_idx, vals, degs):
    c = lax.axis_index("c")
    s = lax.axis_index("s")
    w = s * NC + c
    pltpu.sync_copy(z128, vals)
    r0 = s * RPS
    for k, sz in _zero_chunks(RPS, CH):
        pltpu.sync_copy(vals.at[pl.ds(0, sz)], degs.at[pl.ds(r0 + k, sz)])
    pltpu.sync_copy(o128, vals)
    plsc.subcore_barrier()
    pltpu.sync_copy(dstw.at[w], dst_idx)

    def chunk(i, carry):
        pltpu.sync_copy(vals, degs.at[dst_idx.at[i]], add=True)
        return carry

    lax.fori_loop(0, NCH, chunk, 0)
    plsc.subcore_barrier()
    pltpu.sync_copy(degs.at[pl.ds(r0, RPS)], deg_out.at[c].at[pl.ds(r0, RPS)])


@functools.partial(
    pl.kernel,
    out_type=jax.ShapeDtypeStruct((NC, N_ACC, D), jnp.float32),
    mesh=_mesh,
    scratch_types=[
        pltpu.VMEM((NCH, CH), jnp.int32),
        pltpu.VMEM((NCH, CH), jnp.int32),
        pltpu.VMEM((CH, D), jnp.float32),
        pltpu.VMEM_SHARED((N_ACC, D), jnp.float32),
        pltpu.SemaphoreType.DMA,
    ],
)
def _sc_agg(p_hbm, srcw, dstw, z128, agg_out,
            src_idx, dst_idx, rows, acc, sem):
    c = lax.axis_index("c")
    s = lax.axis_index("s")
    w = s * NC + c
    pltpu.sync_copy(z128, rows)
    r0 = s * RPS
    for k, sz in _zero_chunks(RPS, CH):
        pltpu.sync_copy(rows.at[pl.ds(0, sz)], acc.at[pl.ds(r0 + k, sz)])
    plsc.subcore_barrier()
    pltpu.sync_copy(srcw.at[w], src_idx)
    pltpu.sync_copy(dstw.at[w], dst_idx)

    def chunk(i, carry):
        pltpu.async_copy(p_hbm.at[src_idx.at[i]], rows, sem).wait()
        pltpu.sync_copy(rows, acc.at[dst_idx.at[i]], add=True)
        return carry

    lax.fori_loop(0, NCH, chunk, 0)
    plsc.subcore_barrier()
    pltpu.sync_copy(acc.at[pl.ds(r0, RPS)], agg_out.at[c].at[pl.ds(r0, RPS)])


def _mm1_body(x_ref, w_ref, p_ref, q_ref):
    pq = jax.lax.dot_general(
        x_ref[...], w_ref[...], (((1,), (0,)), ((), ())),
        precision=lax.Precision.HIGHEST, preferred_element_type=jnp.float32)
    p_ref[...] = pq[:, :D]
    q_ref[...] = pq[:, D:]


_mm1 = pl.pallas_call(
    _mm1_body,
    grid=(GRID,),
    in_specs=[
        pl.BlockSpec((BR, D), lambda i: (i, 0)),
        pl.BlockSpec((D, 2 * D), lambda i: (0, 0)),
    ],
    out_specs=[
        pl.BlockSpec((BR, D), lambda i: (i, 0)),
        pl.BlockSpec((BR, D), lambda i: (i, 0)),
    ],
    out_shape=[jax.ShapeDtypeStruct((N, D), jnp.float32),
               jax.ShapeDtypeStruct((N, D), jnp.float32)],
)


def _combine2_body(agg_ref, deg_ref, q_ref, b_ref, w_ref, p2_ref, q2_ref):
    a = agg_ref[0] + agg_ref[1]
    deg = deg_ref[0, :, :1] + deg_ref[1, :, :1]
    mean = a / jnp.maximum(deg, 1.0)
    h = jnp.maximum(mean + b_ref[...] + q_ref[...], 0.0)
    pq = jax.lax.dot_general(
        h, w_ref[...], (((1,), (0,)), ((), ())),
        precision=lax.Precision.HIGHEST, preferred_element_type=jnp.float32)
    p2_ref[...] = pq[:, :D]
    q2_ref[...] = pq[:, D:]


_combine2 = pl.pallas_call(
    _combine2_body,
    grid=(GRID,),
    in_specs=[
        pl.BlockSpec((NC, BR, D), lambda i: (0, i, 0)),
        pl.BlockSpec((NC, BR, D), lambda i: (0, i, 0)),
        pl.BlockSpec((BR, D), lambda i: (i, 0)),
        pl.BlockSpec((1, D), lambda i: (0, 0)),
        pl.BlockSpec((D, 2 * D), lambda i: (0, 0)),
    ],
    out_specs=[
        pl.BlockSpec((BR, D), lambda i: (i, 0)),
        pl.BlockSpec((BR, D), lambda i: (i, 0)),
    ],
    out_shape=[jax.ShapeDtypeStruct((N, D), jnp.float32),
               jax.ShapeDtypeStruct((N, D), jnp.float32)],
)


def _final_body(agg_ref, deg_ref, q_ref, b_ref, o_ref):
    a = agg_ref[0] + agg_ref[1]
    deg = deg_ref[0, :, :1] + deg_ref[1, :, :1]
    mean = a / jnp.maximum(deg, 1.0)
    o_ref[...] = jax.nn.sigmoid(mean + b_ref[...] + q_ref[...])


_final = pl.pallas_call(
    _final_body,
    grid=(GRID,),
    in_specs=[
        pl.BlockSpec((NC, BR, D), lambda i: (0, i, 0)),
        pl.BlockSpec((NC, BR, D), lambda i: (0, i, 0)),
        pl.BlockSpec((BR, D), lambda i: (i, 0)),
        pl.BlockSpec((1, D), lambda i: (0, 0)),
    ],
    out_specs=pl.BlockSpec((BR, D), lambda i: (i, 0)),
    out_shape=jax.ShapeDtypeStruct((N, D), jnp.float32),
)


def kernel(x, edge_index, edge_attr, Wl1, bl1, Wr1, Wl2, bl2, Wr2):
    src = edge_index[0]
    dst = edge_index[1]
    srcw = jnp.concatenate(
        [src, jnp.zeros((EPAD,), jnp.int32)]).reshape(NW, NCH, CH)
    dstw = jnp.concatenate(
        [dst, jnp.full((EPAD,), N, jnp.int32)]).reshape(NW, NCH, CH)
    z128 = jnp.zeros((CH, D), jnp.float32)
    o128 = jnp.ones((CH, D), jnp.float32)
    wcat1 = jnp.concatenate([Wl1.T, Wr1.T], axis=1)
    wcat2 = jnp.concatenate([Wl2.T, Wr2.T], axis=1)
    b1 = bl1.reshape(1, D)
    b2 = bl2.reshape(1, D)

    p1, q1 = _mm1(x, wcat1)
    degp = _sc_deg(dstw, z128, o128)
    agg1 = _sc_agg(p1, srcw, dstw, z128)
    p2, q2 = _combine2(agg1, degp, q1, b1, wcat2)
    agg2 = _sc_agg(p2, srcw, dstw, z128)
    return _final(agg2, degp, q2, b2)
